# baseline (speedup 1.0000x reference)
import functools

import jax
import jax.numpy as jnp
from jax import lax
from jax.experimental import pallas as pl
from jax.experimental.pallas import tpu as pltpu
from jax.experimental.pallas import tpu_sc as plsc

_NUM_CORES = 2
_NUM_SUBCORES = 16
_NUM_WORKERS = _NUM_CORES * _NUM_SUBCORES

_L = 16


def _sc_gather_transposed(ids_t, table):
    S, B = ids_t.shape
    V, D = table.shape
    BBLK = B // _NUM_WORKERS
    NDT = D // 8
    assert BBLK == 128 and D == 64

    mesh = plsc.VectorSubcoreMesh(
        core_axis_name="c", subcore_axis_name="s",
        num_cores=_NUM_CORES, num_subcores=_NUM_SUBCORES)

    SC = 4
    NCH = S // SC
    assert S % SC == 0

    @functools.partial(
        pl.kernel,
        out_type=jax.ShapeDtypeStruct((S, NDT, B // BBLK, 8, BBLK),
                                      jnp.float32),
        mesh=mesh,
        scratch_types=[
            pltpu.VMEM((S, BBLK), jnp.int32),
            pltpu.VMEM((2, SC, BBLK, D), jnp.float32),
            pltpu.VMEM((SC, NDT, 8, BBLK), jnp.float32),
            pltpu.SemaphoreType.DMA,
            pltpu.SemaphoreType.DMA,
            pltpu.SemaphoreType.DMA,
        ],
        compiler_params=pltpu.CompilerParams(
            use_tc_tiling_on_sc=False, needs_layout_passes=False),
    )
    def emb_kernel(ids_hbm, table_hbm, out_hbm, idx_v, rows_v, tbuf_v,
                   gsem0, gsem1, osem):
        wid = lax.axis_index("s") * _NUM_CORES + lax.axis_index("c")
        gsems = (gsem0, gsem1)

        pltpu.sync_copy(ids_hbm.at[:, pl.ds(wid * BBLK, BBLK)], idx_v)

        lane = lax.iota(jnp.int32, _L)
        row_ids = [g * _L + lane for g in range(BBLK // _L)]

        def start_gather(k, p):
            for j in range(SC):
                pltpu.async_copy(
                    table_hbm.at[idx_v.at[k * SC + j]], rows_v.at[p, j],
                    gsems[p])

        def wait_gather(k, p):
            for j in range(SC):
                pltpu.make_async_copy(
                    table_hbm.at[idx_v.at[k * SC + j]], rows_v.at[p, j],
                    gsems[p]).wait()

        def transpose_into(p):
            @plsc.parallel_loop(0, NDT, unroll=2)
            def dt_body(dt):
                for sl in range(SC):
                    for di in range(8):
                        col = jnp.full((_L,), dt * 8 + di, jnp.int32)
                        for g in range(BBLK // _L):
                            vec = plsc.load_gather(
                                rows_v.at[p, sl], [row_ids[g], col])
                            tbuf_v[sl, dt, di, pl.ds(g * _L, _L)] = vec

        def out_copy(k):
            return pltpu.make_async_copy(
                tbuf_v, out_hbm.at[pl.ds(k * SC, SC), :, wid], osem)

        start_gather(0, 0)
        start_gather(1, 1)

        def loop_body(k2, carry):
            for pp in range(2):
                k = 2 * k2 + pp
                wait_gather(k, pp)

                @pl.when(k > 0)
                def _wait_prev_out():
                    out_copy(k).wait()

                transpose_into(pp)
                out_copy(k).start()

                @pl.when(k < NCH - 2)
                def _next_gather():
                    start_gather(k + 2, pp)
            return carry

        lax.fori_loop(0, NCH // 2, loop_body, 0)
        out_copy(NCH - 1).wait()

    return emb_kernel(ids_t, table)


def _tc_workspace(ws_r, bs):
    W = ws_r.shape[0]
    NB = bs // 128

    def body(w_ref, out_ref):
        w2 = w_ref[0]
        out_ref[...] = jnp.broadcast_to(
            w2[None, :, None, :, None], (1, 8, NB, 8, 128))

    return pl.pallas_call(
        body,
        grid=(W,),
        in_specs=[pl.BlockSpec((1, 8, 8), lambda s: (s, 0, 0))],
        out_specs=pl.BlockSpec((1, 8, NB, 8, 128), lambda s: (s, 0, 0, 0, 0)),
        out_shape=jax.ShapeDtypeStruct((W, 8, NB, 8, 128), jnp.float32),
    )(ws_r)


def kernel(input_ids, init_workspace, word_embeddings):
    bs, seq = input_ids.shape
    V, D = word_embeddings.shape
    _, W, H = init_workspace.shape

    ids_t = input_ids.T.astype(jnp.int32)

    ws_r = init_workspace.reshape(W, 8, H // 8)
    ws5 = _tc_workspace(ws_r, bs)
    workspace = ws5.transpose(2, 4, 0, 1, 3).reshape(bs, W, H)

    out5 = _sc_gather_transposed(ids_t, word_embeddings)
    embeddings = out5.transpose(2, 4, 0, 1, 3).reshape(bs, seq, D)
    return (workspace, embeddings)


# speedup vs baseline: 2.4761x; 2.4761x over previous
import functools

import jax
import jax.numpy as jnp
from jax import lax
from jax.experimental import pallas as pl
from jax.experimental.pallas import tpu as pltpu
from jax.experimental.pallas import tpu_sc as plsc

_NUM_CORES = 2
_NUM_SUBCORES = 16
_NUM_WORKERS = _NUM_CORES * _NUM_SUBCORES

_L = 16


def _sc_gather_transposed(ids_t, table):
    S, B = ids_t.shape
    V, D = table.shape
    BBLK = B // _NUM_WORKERS
    NDT = D // 8
    assert BBLK == 128 and D == 64

    mesh = plsc.VectorSubcoreMesh(
        core_axis_name="c", subcore_axis_name="s",
        num_cores=_NUM_CORES, num_subcores=_NUM_SUBCORES)

    SC = 4
    NCH = S // SC
    assert S % SC == 0

    @functools.partial(
        pl.kernel,
        out_type=jax.ShapeDtypeStruct((S, NDT, B // BBLK, 8, BBLK),
                                      jnp.float32),
        mesh=mesh,
        scratch_types=[
            pltpu.VMEM((S, BBLK), jnp.int32),
            pltpu.VMEM((2, SC, BBLK, D), jnp.float32),
            pltpu.VMEM((SC, NDT, 8, BBLK + 1), jnp.float32),
            pltpu.SemaphoreType.DMA,
            pltpu.SemaphoreType.DMA,
            pltpu.SemaphoreType.DMA,
        ],
        compiler_params=pltpu.CompilerParams(
            use_tc_tiling_on_sc=False, needs_layout_passes=False),
    )
    def emb_kernel(ids_hbm, table_hbm, out_hbm, idx_v, rows_v, tbuf_v,
                   gsem0, gsem1, osem):
        wid = lax.axis_index("s") * _NUM_CORES + lax.axis_index("c")
        gsems = (gsem0, gsem1)

        pltpu.sync_copy(ids_hbm.at[:, pl.ds(wid * BBLK, BBLK)], idx_v)

        def start_gather(k, p):
            for j in range(SC):
                pltpu.async_copy(
                    table_hbm.at[idx_v.at[k * SC + j]], rows_v.at[p, j],
                    gsems[p])

        def wait_gather(k, p):
            for j in range(SC):
                pltpu.make_async_copy(
                    table_hbm.at[idx_v.at[k * SC + j]], rows_v.at[p, j],
                    gsems[p]).wait()

        lane = lax.iota(jnp.int32, _L)
        dt_ids = [(q * _L + lane) // 8 for q in range(D // _L)]
        di_ids = [lax.rem(q * _L + lane, 8) for q in range(D // _L)]

        def transpose_into(p):
            @plsc.parallel_loop(0, BBLK, unroll=4)
            def b_body(b):
                bcol = jnp.full((_L,), b, jnp.int32)
                for sl in range(SC):
                    for q in range(D // _L):
                        vec = rows_v[p, sl, b, pl.ds(q * _L, _L)]
                        plsc.store_scatter(
                            tbuf_v.at[sl], [dt_ids[q], di_ids[q], bcol], vec)

        def out_copy(k):
            return pltpu.make_async_copy(
                tbuf_v.at[:, :, :, pl.ds(0, BBLK)],
                out_hbm.at[pl.ds(k * SC, SC), :, wid], osem)

        start_gather(0, 0)
        start_gather(1, 1)

        def loop_body(k2, carry):
            for pp in range(2):
                k = 2 * k2 + pp
                wait_gather(k, pp)

                @pl.when(k > 0)
                def _wait_prev_out():
                    out_copy(k).wait()

                transpose_into(pp)
                out_copy(k).start()

                @pl.when(k < NCH - 2)
                def _next_gather():
                    start_gather(k + 2, pp)
            return carry

        lax.fori_loop(0, NCH // 2, loop_body, 0)
        out_copy(NCH - 1).wait()

    return emb_kernel(ids_t, table)


def _tc_workspace(ws_r, bs):
    W = ws_r.shape[0]
    NB = bs // 128

    def body(w_ref, out_ref):
        w2 = w_ref[0]
        out_ref[...] = jnp.broadcast_to(
            w2[None, :, None, :, None], (1, 8, NB, 8, 128))

    return pl.pallas_call(
        body,
        grid=(W,),
        in_specs=[pl.BlockSpec((1, 8, 8), lambda s: (s, 0, 0))],
        out_specs=pl.BlockSpec((1, 8, NB, 8, 128), lambda s: (s, 0, 0, 0, 0)),
        out_shape=jax.ShapeDtypeStruct((W, 8, NB, 8, 128), jnp.float32),
    )(ws_r)


def kernel(input_ids, init_workspace, word_embeddings):
    bs, seq = input_ids.shape
    V, D = word_embeddings.shape
    _, W, H = init_workspace.shape

    ids_t = input_ids.T.astype(jnp.int32)

    ws_r = init_workspace.reshape(W, 8, H // 8)
    ws5 = _tc_workspace(ws_r, bs)
    workspace = ws5.transpose(2, 4, 0, 1, 3).reshape(bs, W, H)

    out5 = _sc_gather_transposed(ids_t, word_embeddings)
    embeddings = out5.transpose(2, 4, 0, 1, 3).reshape(bs, seq, D)
    return (workspace, embeddings)


# speedup vs baseline: 4.4311x; 1.7895x over previous
import functools

import jax
import jax.numpy as jnp
from jax import lax
from jax.experimental import pallas as pl
from jax.experimental.pallas import tpu as pltpu
from jax.experimental.pallas import tpu_sc as plsc

_NUM_CORES = 2
_NUM_SUBCORES = 16
_NUM_WORKERS = _NUM_CORES * _NUM_SUBCORES

_L = 16


def _sc_gather_transposed(ids_t, table):
    S, B = ids_t.shape
    V, D = table.shape
    BBLK = B // _NUM_WORKERS
    NDT = D // 8
    assert BBLK == 128 and D == 64

    mesh = plsc.VectorSubcoreMesh(
        core_axis_name="c", subcore_axis_name="s",
        num_cores=_NUM_CORES, num_subcores=_NUM_SUBCORES)

    SC = 4
    NCH = S // SC
    assert S % SC == 0

    @functools.partial(
        pl.kernel,
        out_type=jax.ShapeDtypeStruct((S, NDT, B // BBLK, 8, BBLK),
                                      jnp.float32),
        mesh=mesh,
        scratch_types=[
            pltpu.VMEM((S, BBLK), jnp.int32),
            pltpu.VMEM((2, SC, BBLK, D), jnp.float32),
            pltpu.VMEM((SC, NDT, 8, BBLK + 1), jnp.float32),
            pltpu.SemaphoreType.DMA,
            pltpu.SemaphoreType.DMA,
            pltpu.SemaphoreType.DMA,
        ],
        compiler_params=pltpu.CompilerParams(
            use_tc_tiling_on_sc=False, needs_layout_passes=False),
    )
    def emb_kernel(ids_hbm, table_hbm, out_hbm, idx_v, rows_v, tbuf_v,
                   gsem0, gsem1, osem):
        wid = lax.axis_index("s") * _NUM_CORES + lax.axis_index("c")
        gsems = (gsem0, gsem1)

        pltpu.sync_copy(ids_hbm.at[:, pl.ds(wid * BBLK, BBLK)], idx_v)

        def start_gather(k, p):
            for j in range(SC):
                pltpu.async_copy(
                    table_hbm.at[idx_v.at[k * SC + j]], rows_v.at[p, j],
                    gsems[p])

        def wait_gather(k, p):
            for j in range(SC):
                pltpu.make_async_copy(
                    table_hbm.at[idx_v.at[k * SC + j]], rows_v.at[p, j],
                    gsems[p]).wait()

        lane = lax.iota(jnp.int32, _L)
        dt_ids = [(q * _L + lane) // 8 for q in range(D // _L)]
        di_ids = [lax.rem(q * _L + lane, 8) for q in range(D // _L)]

        def transpose_into(p):
            @plsc.parallel_loop(0, BBLK, unroll=4)
            def b_body(b):
                bcol = jnp.full((_L,), b, jnp.int32)
                for sl in range(SC):
                    for q in range(D // _L):
                        vec = rows_v[p, sl, b, pl.ds(q * _L, _L)]
                        plsc.store_scatter(
                            tbuf_v.at[sl], [dt_ids[q], di_ids[q], bcol], vec)

        def out_copy(k):
            return pltpu.make_async_copy(
                tbuf_v.at[:, :, :, pl.ds(0, BBLK)],
                out_hbm.at[pl.ds(k * SC, SC), :, wid], osem)

        start_gather(0, 0)
        start_gather(1, 1)

        def loop_body(k2, carry):
            for pp in range(2):
                k = 2 * k2 + pp
                wait_gather(k, pp)

                @pl.when(k > 0)
                def _wait_prev_out():
                    out_copy(k).wait()

                transpose_into(pp)
                out_copy(k).start()

                @pl.when(k < NCH - 2)
                def _next_gather():
                    start_gather(k + 2, pp)
            return carry

        lax.fori_loop(0, NCH // 2, loop_body, 0)
        out_copy(NCH - 1).wait()

    return emb_kernel(ids_t, table)


_PB = 4096
_HBLK = 122
_H = _HBLK * _PB


def _tc_row_majorize(wt):
    D, V = wt.shape
    G = _HBLK + 1

    def body(a_ref, b_ref, out_ref):
        out_ref[...] = jnp.concatenate(
            [a_ref[...].T, b_ref[...].T], axis=1)

    return pl.pallas_call(
        body,
        grid=(G,),
        in_specs=[pl.BlockSpec((D, _PB), lambda i: (0, i)),
                  pl.BlockSpec((D, _PB), lambda i: (0, i + _HBLK))],
        out_specs=pl.BlockSpec((_PB, 2 * D), lambda i: (i, 0)),
        out_shape=jax.ShapeDtypeStruct((G * _PB, 2 * D), jnp.float32),
    )(wt, wt)


def _tc_workspace(ws_r, bs):
    W = ws_r.shape[0]
    NB = bs // 128

    def body(w_ref, out_ref):
        w2 = w_ref[0]
        out_ref[...] = jnp.broadcast_to(
            w2[None, :, None, :, None], (1, 8, NB, 8, 128))

    return pl.pallas_call(
        body,
        grid=(W,),
        in_specs=[pl.BlockSpec((1, 8, 8), lambda s: (s, 0, 0))],
        out_specs=pl.BlockSpec((1, 8, NB, 8, 128), lambda s: (s, 0, 0, 0, 0)),
        out_shape=jax.ShapeDtypeStruct((W, 8, NB, 8, 128), jnp.float32),
    )(ws_r)


def kernel(input_ids, init_workspace, word_embeddings):
    bs, seq = input_ids.shape
    V, D = word_embeddings.shape
    _, W, H = init_workspace.shape

    ids_t = input_ids.T.astype(jnp.int32)
    pairs = _tc_row_majorize(word_embeddings.T)
    table_rm = pairs.reshape(pairs.shape[0] * 2, D)
    ids_t = 2 * ids_t - jnp.where(ids_t >= _H, 2 * _H - 1, 0)

    ws_r = init_workspace.reshape(W, 8, H // 8)
    ws5 = _tc_workspace(ws_r, bs)
    workspace = ws5.transpose(2, 4, 0, 1, 3).reshape(bs, W, H)

    out5 = _sc_gather_transposed(ids_t, table_rm)
    embeddings = out5.transpose(2, 4, 0, 1, 3).reshape(bs, seq, D)
    return (workspace, embeddings)


# speedup vs baseline: 4.7957x; 1.0823x over previous
import functools

import jax
import jax.numpy as jnp
from jax import lax
from jax.experimental import pallas as pl
from jax.experimental.pallas import tpu as pltpu
from jax.experimental.pallas import tpu_sc as plsc

_NUM_CORES = 2
_NUM_SUBCORES = 16
_NUM_WORKERS = _NUM_CORES * _NUM_SUBCORES

_L = 16


def _sc_gather_transposed(ids_t, table):
    S, B = ids_t.shape
    V, D = table.shape
    BBLK = B // _NUM_WORKERS
    NDT = D // 8
    assert BBLK == 128 and D == 64

    mesh = plsc.VectorSubcoreMesh(
        core_axis_name="c", subcore_axis_name="s",
        num_cores=_NUM_CORES, num_subcores=_NUM_SUBCORES)

    SC = 4
    NCH = S // SC
    assert S % SC == 0

    @functools.partial(
        pl.kernel,
        out_type=jax.ShapeDtypeStruct((S, NDT, B // BBLK, 8, BBLK),
                                      jnp.float32),
        mesh=mesh,
        scratch_types=[
            pltpu.VMEM((S, BBLK), jnp.int32),
            pltpu.VMEM((2, SC, BBLK, D), jnp.float32),
            pltpu.VMEM((SC, NDT, 8, BBLK + 1), jnp.float32),
            pltpu.SemaphoreType.DMA,
            pltpu.SemaphoreType.DMA,
            pltpu.SemaphoreType.DMA,
        ],
        compiler_params=pltpu.CompilerParams(
            use_tc_tiling_on_sc=False, needs_layout_passes=False),
    )
    def emb_kernel(ids_hbm, table_hbm, out_hbm, idx_v, rows_v, tbuf_v,
                   gsem0, gsem1, osem):
        wid = lax.axis_index("s") * _NUM_CORES + lax.axis_index("c")
        gsems = (gsem0, gsem1)

        pltpu.sync_copy(ids_hbm.at[:, pl.ds(wid * BBLK, BBLK)], idx_v)

        def start_gather(k, p):
            for j in range(SC):
                pltpu.async_copy(
                    table_hbm.at[idx_v.at[k * SC + j]], rows_v.at[p, j],
                    gsems[p])

        def wait_gather(k, p):
            for j in range(SC):
                pltpu.make_async_copy(
                    table_hbm.at[idx_v.at[k * SC + j]], rows_v.at[p, j],
                    gsems[p]).wait()

        lane = lax.iota(jnp.int32, _L)
        dt_ids = [(q * _L + lane) // 8 for q in range(D // _L)]
        di_ids = [lax.rem(q * _L + lane, 8) for q in range(D // _L)]

        def transpose_into(p):
            @plsc.parallel_loop(0, BBLK, unroll=4)
            def b_body(b):
                bcol = jnp.full((_L,), b, jnp.int32)
                for sl in range(SC):
                    for q in range(D // _L):
                        vec = rows_v[p, sl, b, pl.ds(q * _L, _L)]
                        plsc.store_scatter(
                            tbuf_v.at[sl], [dt_ids[q], di_ids[q], bcol], vec)

        def out_copy(k):
            return pltpu.make_async_copy(
                tbuf_v.at[:, :, :, pl.ds(0, BBLK)],
                out_hbm.at[pl.ds(k * SC, SC), :, wid], osem)

        start_gather(0, 0)
        start_gather(1, 1)

        def loop_body(k2, carry):
            for pp in range(2):
                k = 2 * k2 + pp
                wait_gather(k, pp)

                @pl.when(k > 0)
                def _wait_prev_out():
                    out_copy(k).wait()

                transpose_into(pp)
                out_copy(k).start()

                @pl.when(k < NCH - 2)
                def _next_gather():
                    start_gather(k + 2, pp)
            return carry

        lax.fori_loop(0, NCH // 2, loop_body, 0)
        out_copy(NCH - 1).wait()

    return emb_kernel(ids_t, table)


_PB = 8192
_HBLK = 61
_H = _HBLK * _PB


def _tc_row_majorize(wt):
    D, V = wt.shape
    G = _HBLK + 1

    def body(a_ref, b_ref, out_ref):
        out_ref[...] = jnp.concatenate(
            [a_ref[...].T, b_ref[...].T], axis=1)

    return pl.pallas_call(
        body,
        grid=(G,),
        in_specs=[pl.BlockSpec((D, _PB), lambda i: (0, i)),
                  pl.BlockSpec((D, _PB), lambda i: (0, i + _HBLK))],
        out_specs=pl.BlockSpec((_PB, 2 * D), lambda i: (i, 0)),
        out_shape=jax.ShapeDtypeStruct((G * _PB, 2 * D), jnp.float32),
    )(wt, wt)


def _tc_workspace(ws_r, bs):
    W = ws_r.shape[0]
    NB = bs // 128

    def body(w_ref, out_ref):
        w2 = w_ref[0]
        out_ref[...] = jnp.broadcast_to(
            w2[None, :, None, :, None], (1, 8, NB, 8, 128))

    return pl.pallas_call(
        body,
        grid=(W,),
        in_specs=[pl.BlockSpec((1, 8, 8), lambda s: (s, 0, 0))],
        out_specs=pl.BlockSpec((1, 8, NB, 8, 128), lambda s: (s, 0, 0, 0, 0)),
        out_shape=jax.ShapeDtypeStruct((W, 8, NB, 8, 128), jnp.float32),
    )(ws_r)


def kernel(input_ids, init_workspace, word_embeddings):
    bs, seq = input_ids.shape
    V, D = word_embeddings.shape
    _, W, H = init_workspace.shape

    ids_t = input_ids.T.astype(jnp.int32)
    pairs = _tc_row_majorize(word_embeddings.T)
    table_rm = pairs.reshape(pairs.shape[0] * 2, D)
    ids_t = 2 * ids_t - jnp.where(ids_t >= _H, 2 * _H - 1, 0)

    ws_r = init_workspace.reshape(W, 8, H // 8)
    ws5 = _tc_workspace(ws_r, bs)
    workspace = ws5.transpose(2, 4, 0, 1, 3).reshape(bs, W, H)

    out5 = _sc_gather_transposed(ids_t, table_rm)
    embeddings = out5.transpose(2, 4, 0, 1, 3).reshape(bs, seq, D)
    return (workspace, embeddings)


# speedup vs baseline: 4.8740x; 1.0163x over previous
import functools

import jax
import jax.numpy as jnp
from jax import lax
from jax.experimental import pallas as pl
from jax.experimental.pallas import tpu as pltpu
from jax.experimental.pallas import tpu_sc as plsc

_NUM_CORES = 2
_NUM_SUBCORES = 16
_NUM_WORKERS = _NUM_CORES * _NUM_SUBCORES

_L = 16


def _sc_gather_transposed(ids_t, table):
    S, B = ids_t.shape
    V, D = table.shape
    BBLK = B // _NUM_WORKERS
    NDT = D // 8
    assert BBLK == 128 and D == 64

    mesh = plsc.VectorSubcoreMesh(
        core_axis_name="c", subcore_axis_name="s",
        num_cores=_NUM_CORES, num_subcores=_NUM_SUBCORES)

    SC = 4
    NCH = S // SC
    assert S % SC == 0

    @functools.partial(
        pl.kernel,
        out_type=jax.ShapeDtypeStruct((S, NDT, B // BBLK, 8, BBLK),
                                      jnp.float32),
        mesh=mesh,
        scratch_types=[
            pltpu.VMEM((S, BBLK), jnp.int32),
            pltpu.VMEM((2, SC, BBLK, D), jnp.float32),
            pltpu.VMEM((SC, NDT, 8, BBLK + 1), jnp.float32),
            pltpu.SemaphoreType.DMA,
            pltpu.SemaphoreType.DMA,
            pltpu.SemaphoreType.DMA,
        ],
        compiler_params=pltpu.CompilerParams(
            use_tc_tiling_on_sc=False, needs_layout_passes=False),
    )
    def emb_kernel(ids_hbm, table_hbm, out_hbm, idx_v, rows_v, tbuf_v,
                   gsem0, gsem1, osem):
        wid = lax.axis_index("s") * _NUM_CORES + lax.axis_index("c")
        gsems = (gsem0, gsem1)

        pltpu.sync_copy(ids_hbm.at[:, pl.ds(wid * BBLK, BBLK)], idx_v)

        def start_gather(k, p):
            for j in range(SC):
                pltpu.async_copy(
                    table_hbm.at[idx_v.at[k * SC + j]], rows_v.at[p, j],
                    gsems[p])

        def wait_gather(k, p):
            for j in range(SC):
                pltpu.make_async_copy(
                    table_hbm.at[idx_v.at[k * SC + j]], rows_v.at[p, j],
                    gsems[p]).wait()

        lane = lax.iota(jnp.int32, _L)
        dt_ids = [(q * _L + lane) // 8 for q in range(D // _L)]
        di_ids = [lax.rem(q * _L + lane, 8) for q in range(D // _L)]

        def transpose_into(p):
            @plsc.parallel_loop(0, BBLK, unroll=4)
            def b_body(b):
                bcol = jnp.full((_L,), b, jnp.int32)
                for sl in range(SC):
                    for q in range(D // _L):
                        vec = rows_v[p, sl, b, pl.ds(q * _L, _L)]
                        plsc.store_scatter(
                            tbuf_v.at[sl], [dt_ids[q], di_ids[q], bcol], vec)

        def out_copy(k):
            return pltpu.make_async_copy(
                tbuf_v.at[:, :, :, pl.ds(0, BBLK)],
                out_hbm.at[pl.ds(k * SC, SC), :, wid], osem)

        start_gather(0, 0)
        start_gather(1, 1)

        def loop_body(k2, carry):
            for pp in range(2):
                k = 2 * k2 + pp
                wait_gather(k, pp)

                @pl.when(k > 0)
                def _wait_prev_out():
                    out_copy(k).wait()

                transpose_into(pp)
                out_copy(k).start()

                @pl.when(k < NCH - 2)
                def _next_gather():
                    start_gather(k + 2, pp)
            return carry

        lax.fori_loop(0, NCH // 2, loop_body, 0)
        out_copy(NCH - 1).wait()

    return emb_kernel(ids_t, table)


_PB = 16384
_HBLK = 30
_H = _HBLK * _PB


def _tc_row_majorize(wt):
    D, V = wt.shape
    G = _HBLK + 2

    def body(a_ref, b_ref, out_ref):
        out_ref[...] = jnp.concatenate(
            [a_ref[...].T, b_ref[...].T], axis=1)

    return pl.pallas_call(
        body,
        grid=(G,),
        in_specs=[pl.BlockSpec((D, _PB), lambda i: (0, i)),
                  pl.BlockSpec((D, _PB), lambda i: (0, i + _HBLK))],
        out_specs=pl.BlockSpec((_PB, 2 * D), lambda i: (i, 0)),
        out_shape=jax.ShapeDtypeStruct((G * _PB, 2 * D), jnp.float32),
    )(wt, wt)


def _tc_workspace(ws_r, bs):
    W = ws_r.shape[0]
    NB = bs // 128

    def body(w_ref, out_ref):
        w2 = w_ref[0]
        out_ref[...] = jnp.broadcast_to(
            w2[None, :, None, :, None], (1, 8, NB, 8, 128))

    return pl.pallas_call(
        body,
        grid=(W,),
        in_specs=[pl.BlockSpec((1, 8, 8), lambda s: (s, 0, 0))],
        out_specs=pl.BlockSpec((1, 8, NB, 8, 128), lambda s: (s, 0, 0, 0, 0)),
        out_shape=jax.ShapeDtypeStruct((W, 8, NB, 8, 128), jnp.float32),
    )(ws_r)


def kernel(input_ids, init_workspace, word_embeddings):
    bs, seq = input_ids.shape
    V, D = word_embeddings.shape
    _, W, H = init_workspace.shape

    ids_t = input_ids.T.astype(jnp.int32)
    pairs = _tc_row_majorize(word_embeddings.T)
    table_rm = pairs.reshape(pairs.shape[0] * 2, D)
    ids_t = 2 * ids_t - jnp.where(ids_t >= _H, 2 * _H - 1, 0)

    ws_r = init_workspace.reshape(W, 8, H // 8)
    ws5 = _tc_workspace(ws_r, bs)
    workspace = ws5.transpose(2, 4, 0, 1, 3).reshape(bs, W, H)

    out5 = _sc_gather_transposed(ids_t, table_rm)
    embeddings = out5.transpose(2, 4, 0, 1, 3).reshape(bs, seq, D)
    return (workspace, embeddings)
